# Initial kernel scaffold; baseline (speedup 1.0000x reference)
import jax
import jax.numpy as jnp
from jax.experimental import pallas as pl


def kernel(x, pkm_keys, expert_w, expert_b, proj_w, proj_b):
    raise NotImplementedError("write your pallas kernel here")


# speedup vs baseline: 11.3345x; 11.3345x over previous
import functools

import jax
import jax.numpy as jnp
import numpy as np
from jax.experimental import pallas as pl
from jax.experimental.pallas import tpu as pltpu

_T = 2048
_D = 64
_NH = 8
_NK = 32
_DH = 8
_NE = _NH * _NK
_RB = _T // _NH
_K = 32

_DN = (((1,), (1,)), ((), ()))
_DNM = (((1,), (0,)), ((), ()))


def _body(xr_ref, kb_ref, xf_ref, w_ref, b_ref, pw_ref, pb_ref, o_ref):
    f32 = jnp.float32
    xr = xr_ref[0]
    kb = kb_ref[0]
    scores = jax.lax.dot_general(xr, kb, _DN, preferred_element_type=f32)

    bits = jax.lax.bitcast_convert_type(scores, jnp.int32)
    key = bits ^ ((bits >> 31) & jnp.int32(0x7FFFFFFF))

    sign = jnp.int32(-(2**31))
    p_u = jnp.zeros((_RB, 1), jnp.int32)
    kf = jnp.float32(_K)
    for bit in range(31, -1, -1):
        mask = jnp.int32(-(2**31)) if bit == 31 else jnp.int32(1 << bit)
        cand_u = p_u | mask
        cand_s = cand_u ^ sign
        cnt = jnp.sum((key >= cand_s).astype(f32), axis=1, keepdims=True)
        p_u = jnp.where(cnt >= kf, cand_u, p_u)
    t_s = p_u ^ sign

    gt = key > t_s
    eq = key == t_s
    gtf = gt.astype(f32)
    eqf = eq.astype(f32)
    ng = jnp.sum(gtf, axis=1, keepdims=True)
    ri = jax.lax.broadcasted_iota(jnp.int32, (_NE, _NE), 0)
    ci = jax.lax.broadcasted_iota(jnp.int32, (_NE, _NE), 1)
    lt = (ri < ci).astype(f32)
    prefix = jax.lax.dot_general(eqf, lt, _DNM, preferred_element_type=f32)
    sel = gtf + eqf * (prefix < (kf - ng)).astype(f32)

    m = jnp.max(scores, axis=1, keepdims=True)
    e = jnp.exp(scores - m) * sel
    z = jnp.sum(e, axis=1, keepdims=True)
    p = e / z

    s1 = jax.lax.dot_general(xf_ref[...], w_ref[...], _DN,
                             preferred_element_type=f32)
    c = jnp.sum(p * s1, axis=1, keepdims=True)
    bp = jax.lax.dot_general(b_ref[...], pw_ref[...], _DN,
                             preferred_element_type=f32)
    ones = jnp.ones((1, _D), f32)
    rp = jax.lax.dot_general(ones, pw_ref[...], _DN,
                             preferred_element_type=f32)
    out = jax.lax.dot_general(p, bp, _DNM, preferred_element_type=f32)
    o_ref[...] = out + c * rp + pb_ref[...]


@functools.partial(jax.jit, static_argnames=())
def kernel(x, pkm_keys, expert_w, expert_b, proj_w, proj_b):
    assert x.shape == (1, _T, _D) and pkm_keys.shape == (_NH, _NK, _DH)
    xf = x.reshape(_T, _D)
    xr = xf.reshape(_RB, _NH, _NH, _DH).transpose(2, 0, 1, 3).reshape(_NH, _RB, _D)
    eye = jnp.eye(_NH, dtype=pkm_keys.dtype)
    kb = (pkm_keys[:, None, :, None, :] * eye[None, :, None, :, None]).reshape(
        _NH, _NE, _D)
    w256 = expert_w[:_NE]
    b256 = expert_b[:_NE]
    pb2 = proj_b.reshape(1, _D)

    out = pl.pallas_call(
        _body,
        grid=(_NH,),
        in_specs=[
            pl.BlockSpec((1, _RB, _D), lambda h: (h, 0, 0)),
            pl.BlockSpec((1, _NE, _D), lambda h: (h, 0, 0)),
            pl.BlockSpec((_RB, _D), lambda h: (h, 0)),
            pl.BlockSpec((_NE, _D), lambda h: (0, 0)),
            pl.BlockSpec((_NE, _D), lambda h: (0, 0)),
            pl.BlockSpec((_D, _D), lambda h: (0, 0)),
            pl.BlockSpec((1, _D), lambda h: (0, 0)),
        ],
        out_specs=pl.BlockSpec((_RB, _D), lambda h: (h, 0)),
        out_shape=jax.ShapeDtypeStruct((_T, _D), jnp.float32),
    )(xr, kb, xf, w256, b256, proj_w, pb2)
    return out.reshape(1, _T, _D)


# speedup vs baseline: 13.2531x; 1.1693x over previous
import functools

import jax
import jax.numpy as jnp
import numpy as np
from jax.experimental import pallas as pl
from jax.experimental.pallas import tpu as pltpu

_T = 2048
_D = 64
_NH = 8
_NK = 32
_DH = 8
_NE = _NH * _NK
_RB = _T // _NH
_K = 32

_DT = (((1,), (1,)), ((), ()))
_DM = (((1,), (0,)), ((), ()))


def _body(xrt_ref, kb_ref, xt_ref, w_ref, b_ref, pw_ref, pb_ref, o_ref):
    f32 = jnp.float32
    st = jax.lax.dot_general(kb_ref[0], xrt_ref[0], _DM,
                             preferred_element_type=f32)

    bits = jax.lax.bitcast_convert_type(st, jnp.int32)
    key = bits ^ ((bits >> 31) & jnp.int32(0x7FFFFFFF))

    sign = jnp.int32(-(2**31))
    p_u = jnp.zeros((1, _RB), jnp.int32)
    kf = jnp.float32(_K)
    for bit in range(31, -1, -1):
        mask = jnp.int32(-(2**31)) if bit == 31 else jnp.int32(1 << bit)
        cand_u = p_u | mask
        cand_s = cand_u ^ sign
        cnt = jnp.sum((key >= cand_s).astype(f32), axis=0, keepdims=True)
        p_u = jnp.where(cnt >= kf, cand_u, p_u)
    t_s = p_u ^ sign

    gt = key > t_s
    eq = key == t_s
    gtf = gt.astype(f32)
    eqf = eq.astype(f32)
    ng = jnp.sum(gtf, axis=0, keepdims=True)
    ri = jax.lax.broadcasted_iota(jnp.int32, (_NE, _NE), 0)
    ci = jax.lax.broadcasted_iota(jnp.int32, (_NE, _NE), 1)
    lt = (ri > ci).astype(f32)
    prefix = jax.lax.dot_general(lt, eqf, _DM, preferred_element_type=f32)
    sel = gtf + eqf * (prefix < (kf - ng)).astype(f32)

    m = jnp.max(st, axis=0, keepdims=True)
    e = jnp.exp(st - m) * sel
    z = jnp.sum(e, axis=0, keepdims=True)
    pt = e * (1.0 / z)

    s1t = jax.lax.dot_general(w_ref[...], xt_ref[...], _DM,
                              preferred_element_type=f32)
    acat = jnp.concatenate([pt, pt * s1t], axis=0)
    bpt = jax.lax.dot_general(pw_ref[...], b_ref[...], _DT,
                              preferred_element_type=f32)
    ones = jnp.ones((1, _D), f32)
    rpt = jax.lax.dot_general(pw_ref[...], ones, _DT,
                              preferred_element_type=f32)
    bcat = jnp.concatenate(
        [bpt, jnp.broadcast_to(rpt, (_D, _NE))], axis=1)
    outt = jax.lax.dot_general(bcat, acat, _DM,
                               preferred_element_type=f32)
    ident = (ri == ci).astype(f32)
    out = jax.lax.dot_general(ident, outt, _DT,
                              preferred_element_type=f32)
    o_ref[...] = out + pb_ref[...]


@functools.partial(jax.jit, static_argnames=())
def kernel(x, pkm_keys, expert_w, expert_b, proj_w, proj_b):
    assert x.shape == (1, _T, _D) and pkm_keys.shape == (_NH, _NK, _DH)
    xf = x.reshape(_T, _D)
    xt = xf.T
    xrt = xt.reshape(_NH, _DH, _RB, _NH).transpose(0, 3, 1, 2).reshape(
        _NH, _D, _RB)
    eye = jnp.eye(_NH, dtype=pkm_keys.dtype)
    kb = (pkm_keys[:, None, :, None, :] * eye[None, :, None, :, None]).reshape(
        _NH, _NE, _D)
    w256 = expert_w[:_NE]
    b256 = expert_b[:_NE]
    pb2 = proj_b.reshape(1, _D)

    out = pl.pallas_call(
        _body,
        grid=(_NH,),
        in_specs=[
            pl.BlockSpec((1, _D, _RB), lambda h: (h, 0, 0)),
            pl.BlockSpec((1, _NE, _D), lambda h: (h, 0, 0)),
            pl.BlockSpec((_D, _RB), lambda h: (0, h)),
            pl.BlockSpec((_NE, _D), lambda h: (0, 0)),
            pl.BlockSpec((_NE, _D), lambda h: (0, 0)),
            pl.BlockSpec((_D, _D), lambda h: (0, 0)),
            pl.BlockSpec((1, _D), lambda h: (0, 0)),
        ],
        out_specs=pl.BlockSpec((_RB, _D), lambda h: (h, 0)),
        out_shape=jax.ShapeDtypeStruct((_T, _D), jnp.float32),
    )(xrt, kb, xt, w256, b256, proj_w, pb2)
    return out.reshape(1, _T, _D)
